# Initial kernel scaffold; baseline (speedup 1.0000x reference)
import functools

import jax
import jax.numpy as jnp
from jax import lax
from jax.experimental import pallas as pl
from jax.experimental.pallas import tpu as pltpu
from jax.experimental.pallas import tpu_sc as plsc

N_NODES = 10000
N_PAD = 10240
D_HID = 16
N_EDGES = 320000
NC = 2
NS = 16
NW = NC * NS
CHUNK = 80
EPT = N_EDGES // NW
NCHUNK = EPT // CHUNK
RPS = N_PAD // NS


def _sc_mesh():
  return plsc.VectorSubcoreMesh(core_axis_name="c", subcore_axis_name="s")


def _sc_deg(dst2d):
  @functools.partial(
      pl.kernel,
      out_type=jax.ShapeDtypeStruct((NW, N_PAD), jnp.float32),
      mesh=_sc_mesh(),
      scratch_types=[
          pltpu.VMEM((NCHUNK, CHUNK), jnp.int32),
          pltpu.VMEM((N_PAD,), jnp.float32),
      ],
  )
  def k(dst_hbm, out_hbm, dst_v, hist_v):
    c = lax.axis_index("c")
    s = lax.axis_index("s")
    wid = c * NS + s
    zero16 = jnp.zeros((16,), jnp.float32)

    @pl.loop(0, N_PAD, step=16)
    def _(i):
      hist_v[pl.ds(i, 16)] = zero16

    pltpu.sync_copy(dst_hbm.at[pl.ds(wid * NCHUNK, NCHUNK)], dst_v)

    @pl.loop(0, NCHUNK)
    def _(j):
      @pl.loop(0, CHUNK, step=16)
      def _(t):
        idx = dst_v[j, pl.ds(t, 16)]
        cnt, last = plsc.scan_count(idx)
        val = (cnt + 1).astype(jnp.float32)
        plsc.addupdate_scatter(hist_v, [idx], val, mask=last)

    pltpu.sync_copy(hist_v, out_hbm.at[wid])

  return k(dst2d)


def _sc_agg(table, src2d, dst2d):
  @functools.partial(
      pl.kernel,
      out_type=jax.ShapeDtypeStruct((NC, N_PAD, D_HID), jnp.float32),
      mesh=_sc_mesh(),
      scratch_types=[
          pltpu.VMEM((NCHUNK, CHUNK), jnp.int32),
          pltpu.VMEM((NCHUNK, CHUNK), jnp.int32),
          pltpu.VMEM((CHUNK, D_HID), jnp.float32),
          pltpu.VMEM((RPS, D_HID), jnp.float32),
          pltpu.VMEM_SHARED((N_PAD, D_HID), jnp.float32),
          pltpu.SemaphoreType.DMA,
      ],
  )
  def k(tab_hbm, src_hbm, dst_hbm, out_hbm, src_v, dst_v, rows_v, stage_v,
        acc_sh, sem):
    c = lax.axis_index("c")
    s = lax.axis_index("s")
    wid = c * NS + s
    zero16 = jnp.zeros((16,), jnp.float32)

    @pl.loop(0, RPS)
    def _(i):
      stage_v[i, :] = zero16

    pltpu.sync_copy(stage_v, acc_sh.at[pl.ds(s * RPS, RPS)])
    plsc.subcore_barrier()

    pltpu.sync_copy(src_hbm.at[pl.ds(wid * NCHUNK, NCHUNK)], src_v)
    pltpu.sync_copy(dst_hbm.at[pl.ds(wid * NCHUNK, NCHUNK)], dst_v)

    @pl.loop(0, NCHUNK)
    def _(j):
      pltpu.async_copy(tab_hbm.at[src_v.at[j]], rows_v, sem).wait()
      pltpu.sync_copy(rows_v, acc_sh.at[dst_v.at[j]], add=True)

    plsc.subcore_barrier()
    pltpu.sync_copy(acc_sh.at[pl.ds(s * RPS, RPS)], stage_v)
    pltpu.sync_copy(stage_v, out_hbm.at[c].at[pl.ds(s * RPS, RPS)])

  return k(table, src2d, dst2d)


def _tc_mm(x_pad, w1):
  def body(x_ref, w_ref, o_ref):
    o_ref[...] = jnp.dot(x_ref[...], w_ref[...],
                         preferred_element_type=jnp.float32)

  return pl.pallas_call(
      body,
      out_shape=jax.ShapeDtypeStruct((N_PAD, D_HID), jnp.float32),
  )(x_pad, w1)


def _tc_prep(hist, xw1):
  def body(h_ref, xw_ref, dis_ref, tab_ref):
    ones = jnp.ones((NW, D_HID), jnp.float32)
    deg = lax.dot_general(h_ref[...], ones, (((0,), (0,)), ((), ())),
                          preferred_element_type=jnp.float32) + 1.0
    dis = lax.rsqrt(deg)
    dis_ref[...] = dis
    tab_ref[...] = dis * xw_ref[...]

  return pl.pallas_call(
      body,
      out_shape=[
          jax.ShapeDtypeStruct((N_PAD, D_HID), jnp.float32),
          jax.ShapeDtypeStruct((N_PAD, D_HID), jnp.float32),
      ],
  )(hist, xw1)


def _tc_mid(p1, dis16, table1, b1):
  def body(p_ref, dis_ref, tab_ref, b_ref, o_ref):
    dis = dis_ref[...]
    h = jnp.maximum(dis * (p_ref[0] + p_ref[1] + tab_ref[...]) + b_ref[...],
                    0.0)
    o_ref[...] = dis * h

  return pl.pallas_call(
      body,
      out_shape=jax.ShapeDtypeStruct((N_PAD, D_HID), jnp.float32),
  )(p1, dis16, table1, b1)


def _tc_final(p2, dis16, table2, w2, b2):
  def body(p_ref, dis_ref, tab_ref, w_ref, b_ref, o_ref):
    ah = dis_ref[...] * (p_ref[0] + p_ref[1] + tab_ref[...])
    o_ref[...] = jnp.dot(ah, w_ref[...],
                         preferred_element_type=jnp.float32) + b_ref[...]

  return pl.pallas_call(
      body,
      out_shape=jax.ShapeDtypeStruct((N_PAD, w2.shape[1]), jnp.float32),
  )(p2, dis16, table2, w2, b2)


@jax.jit
def kernel(x, edge_index, W1, b1, W2, b2):
  x_pad = jnp.pad(x, ((0, N_PAD - N_NODES), (0, 0)))
  src2 = edge_index[0].reshape(NW * NCHUNK, CHUNK)
  dst2 = edge_index[1].reshape(NW * NCHUNK, CHUNK)

  xw1 = _tc_mm(x_pad, W1)
  hist = _sc_deg(dst2)
  dis16, table1 = _tc_prep(hist, xw1)
  p1 = _sc_agg(table1, src2, dst2)
  table2 = _tc_mid(p1, dis16, table1, b1.reshape(1, D_HID))
  p2 = _sc_agg(table2, src2, dst2)
  out = _tc_final(p2, dis16, table2, W2, b2.reshape(1, -1))
  return out[:N_NODES]


# speedup vs baseline: 32.0594x; 32.0594x over previous
import functools

import jax
import jax.numpy as jnp
from jax import lax
from jax.experimental import pallas as pl
from jax.experimental.pallas import tpu as pltpu
from jax.experimental.pallas import tpu_sc as plsc

N_NODES = 10000
N_PAD = 10240
D_HID = 16
N_EDGES = 320000
NC = 2
NS = 16
NW = NC * NS
CHUNK = 128
NCHUNK = 80
EPT = CHUNK * NCHUNK
E_PAD = NW * EPT
RPS = N_PAD // NS


def _sc_mesh():
  return plsc.VectorSubcoreMesh(core_axis_name="c", subcore_axis_name="s")


_SC_PARAMS = pltpu.CompilerParams(needs_layout_passes=False,
                                  use_tc_tiling_on_sc=False)


def _sc_deg(dst1d):
  @functools.partial(
      pl.kernel,
      out_type=jax.ShapeDtypeStruct((NW * N_PAD,), jnp.float32),
      mesh=_sc_mesh(),
      compiler_params=_SC_PARAMS,
      scratch_types=[
          pltpu.VMEM((EPT,), jnp.int32),
          pltpu.VMEM((N_PAD,), jnp.float32),
      ],
  )
  def k(dst_hbm, out_hbm, dst_v, hist_v):
    c = lax.axis_index("c")
    s = lax.axis_index("s")
    wid = c * NS + s
    zero16 = jnp.zeros((16,), jnp.float32)

    @pl.loop(0, N_PAD, step=16)
    def _(i):
      hist_v[pl.ds(i, 16)] = zero16

    pltpu.sync_copy(dst_hbm.at[pl.ds(wid * EPT, EPT)], dst_v)

    @pl.loop(0, EPT, step=16)
    def _(t):
      idx = dst_v[pl.ds(t, 16)]
      cnt, last = plsc.scan_count(idx)
      plsc.addupdate_scatter(hist_v, [idx], cnt.astype(jnp.float32), mask=last)

    pltpu.sync_copy(hist_v, out_hbm.at[pl.ds(wid * N_PAD, N_PAD)])

  return k(dst1d)


def _sc_agg(table, src2d, dst2d):
  @functools.partial(
      pl.kernel,
      out_type=jax.ShapeDtypeStruct((NC, N_PAD, D_HID), jnp.float32),
      mesh=_sc_mesh(),
      compiler_params=_SC_PARAMS,
      scratch_types=[
          pltpu.VMEM((NCHUNK, CHUNK), jnp.int32),
          pltpu.VMEM((NCHUNK, CHUNK), jnp.int32),
          pltpu.VMEM((CHUNK, D_HID), jnp.float32),
          pltpu.VMEM((RPS, D_HID), jnp.float32),
          pltpu.VMEM_SHARED((N_PAD, D_HID), jnp.float32),
          pltpu.SemaphoreType.DMA,
      ],
  )
  def k(tab_hbm, src_hbm, dst_hbm, out_hbm, src_v, dst_v, rows_v, stage_v,
        acc_sh, sem):
    c = lax.axis_index("c")
    s = lax.axis_index("s")
    wid = c * NS + s
    zero16 = jnp.zeros((16,), jnp.float32)

    @pl.loop(0, RPS)
    def _(i):
      stage_v[i, :] = zero16

    pltpu.sync_copy(stage_v, acc_sh.at[pl.ds(s * RPS, RPS)])
    plsc.subcore_barrier()

    pltpu.sync_copy(src_hbm.at[pl.ds(wid * NCHUNK, NCHUNK)], src_v)
    pltpu.sync_copy(dst_hbm.at[pl.ds(wid * NCHUNK, NCHUNK)], dst_v)

    @pl.loop(0, NCHUNK)
    def _(j):
      pltpu.async_copy(tab_hbm.at[src_v.at[j]], rows_v, sem).wait()
      pltpu.sync_copy(rows_v, acc_sh.at[dst_v.at[j]], add=True)

    plsc.subcore_barrier()
    pltpu.sync_copy(acc_sh.at[pl.ds(s * RPS, RPS)], stage_v)
    pltpu.sync_copy(stage_v, out_hbm.at[c].at[pl.ds(s * RPS, RPS)])

  return k(table, src2d, dst2d)


def _tc_mm(x_pad, w1):
  def body(x_ref, w_ref, o_ref):
    o_ref[...] = jnp.dot(x_ref[...], w_ref[...],
                         preferred_element_type=jnp.float32)

  return pl.pallas_call(
      body,
      out_shape=jax.ShapeDtypeStruct((N_PAD, D_HID), jnp.float32),
  )(x_pad, w1)


def _tc_prep(hist, xw1):
  def body(h_ref, xw_ref, dis_ref, tab_ref):
    ones = jnp.ones((NW, D_HID), jnp.float32)
    deg = lax.dot_general(h_ref[...], ones, (((0,), (0,)), ((), ())),
                          preferred_element_type=jnp.float32) + 1.0
    dis = lax.rsqrt(deg)
    dis_ref[...] = dis
    tab_ref[...] = dis * xw_ref[...]

  return pl.pallas_call(
      body,
      out_shape=[
          jax.ShapeDtypeStruct((N_PAD, D_HID), jnp.float32),
          jax.ShapeDtypeStruct((N_PAD, D_HID), jnp.float32),
      ],
  )(hist, xw1)


def _tc_mid(p1, dis16, table1, b1):
  def body(p_ref, dis_ref, tab_ref, b_ref, o_ref):
    dis = dis_ref[...]
    h = jnp.maximum(dis * (p_ref[0] + p_ref[1] + tab_ref[...]) + b_ref[...],
                    0.0)
    o_ref[...] = dis * h

  return pl.pallas_call(
      body,
      out_shape=jax.ShapeDtypeStruct((N_PAD, D_HID), jnp.float32),
  )(p1, dis16, table1, b1)


def _tc_final(p2, dis16, table2, w2, b2):
  def body(p_ref, dis_ref, tab_ref, w_ref, b_ref, o_ref):
    ah = dis_ref[...] * (p_ref[0] + p_ref[1] + tab_ref[...])
    o_ref[...] = jnp.dot(ah, w_ref[...],
                         preferred_element_type=jnp.float32) + b_ref[...]

  return pl.pallas_call(
      body,
      out_shape=jax.ShapeDtypeStruct((N_PAD, w2.shape[1]), jnp.float32),
  )(p2, dis16, table2, w2, b2)


@jax.jit
def kernel(x, edge_index, W1, b1, W2, b2):
  x_pad = jnp.pad(x, ((0, N_PAD - N_NODES), (0, 0)))
  pad = jnp.full((2, E_PAD - N_EDGES), N_PAD - 1, dtype=edge_index.dtype)
  edges = jnp.concatenate([edge_index, pad], axis=1)
  src2 = edges[0].reshape(NW * NCHUNK, CHUNK)
  dst2 = edges[1].reshape(NW * NCHUNK, CHUNK)

  xw1 = _tc_mm(x_pad, W1)
  hist = _sc_deg(edges[1]).reshape(NW, N_PAD)
  dis16, table1 = _tc_prep(hist, xw1)
  p1 = _sc_agg(table1, src2, dst2)
  table2 = _tc_mid(p1, dis16, table1, b1.reshape(1, D_HID))
  p2 = _sc_agg(table2, src2, dst2)
  out = _tc_final(p2, dis16, table2, W2, b2.reshape(1, -1))
  return out[:N_NODES]


# speedup vs baseline: 45.4249x; 1.4169x over previous
import functools

import jax
import jax.numpy as jnp
from jax import lax
from jax.experimental import pallas as pl
from jax.experimental.pallas import tpu as pltpu
from jax.experimental.pallas import tpu_sc as plsc

N_NODES = 10000
N_PAD = 10240
D_HID = 16
N_EDGES = 320000
NC = 2
NS = 16
NW = NC * NS
CHUNK = 128
NCHUNK = 80
EPT = CHUNK * NCHUNK
E_PAD = NW * EPT
RPS = N_PAD // NS
D_OUT = 3


def _sc_mesh():
  return plsc.VectorSubcoreMesh(core_axis_name="c", subcore_axis_name="s")


_SC_PARAMS = pltpu.CompilerParams(needs_layout_passes=False,
                                  use_tc_tiling_on_sc=False)


def _sc_deg(dst1d):
  @functools.partial(
      pl.kernel,
      out_type=jax.ShapeDtypeStruct((NW * N_PAD,), jnp.float32),
      mesh=_sc_mesh(),
      compiler_params=_SC_PARAMS,
      scratch_types=[
          pltpu.VMEM((EPT,), jnp.int32),
          pltpu.VMEM((N_PAD,), jnp.float32),
      ],
  )
  def k(dst_hbm, out_hbm, dst_v, hist_v):
    c = lax.axis_index("c")
    s = lax.axis_index("s")
    wid = c * NS + s
    zero16 = jnp.zeros((16,), jnp.float32)

    @pl.loop(0, N_PAD, step=16)
    def _(i):
      hist_v[pl.ds(i, 16)] = zero16

    pltpu.sync_copy(dst_hbm.at[pl.ds(wid * EPT, EPT)], dst_v)

    @pl.loop(0, EPT, step=16)
    def _(t):
      idx = dst_v[pl.ds(t, 16)]
      cnt, last = plsc.scan_count(idx)
      plsc.addupdate_scatter(hist_v, [idx], cnt.astype(jnp.float32), mask=last)

    pltpu.sync_copy(hist_v, out_hbm.at[pl.ds(wid * N_PAD, N_PAD)])

  return k(dst1d)


KB = 4
NBLK = NCHUNK // KB


def _sc_agg(table, src2d, dst2d, zeros, w):
  @functools.partial(
      pl.kernel,
      out_type=jax.ShapeDtypeStruct((NC, N_PAD, w), jnp.float32),
      mesh=_sc_mesh(),
      compiler_params=_SC_PARAMS,
      scratch_types=[
          pltpu.VMEM((NCHUNK, CHUNK), jnp.int32),
          pltpu.VMEM((NCHUNK, CHUNK), jnp.int32),
          pltpu.VMEM((KB, CHUNK, w), jnp.float32),
          pltpu.VMEM((KB, CHUNK, w), jnp.float32),
          pltpu.VMEM_SHARED((N_PAD, w), jnp.float32),
          pltpu.SemaphoreType.DMA,
          pltpu.SemaphoreType.DMA,
      ],
  )
  def k(tab_hbm, src_hbm, dst_hbm, z_hbm, out_hbm, src_v, dst_v, buf_a, buf_b,
        acc_sh, sem_a, sem_b):
    c = lax.axis_index("c")
    s = lax.axis_index("s")
    wid = c * NS + s

    pltpu.sync_copy(z_hbm.at[pl.ds(s * RPS, RPS)], acc_sh.at[pl.ds(s * RPS, RPS)])
    pltpu.sync_copy(src_hbm.at[pl.ds(wid * NCHUNK, NCHUNK)], src_v)
    pltpu.sync_copy(dst_hbm.at[pl.ds(wid * NCHUNK, NCHUNK)], dst_v)
    plsc.subcore_barrier()

    def fire_g(b, bufs, sem):
      for t in range(KB):
        pltpu.async_copy(tab_hbm.at[src_v.at[b * KB + t]], bufs.at[t], sem)

    def drain_g(bufs, sem):
      for t in range(KB):
        pltpu.make_async_copy(tab_hbm.at[src_v.at[0]], bufs.at[t], sem).wait()

    def scatter(b, bufs):
      for t in range(KB):
        pltpu.sync_copy(bufs.at[t], acc_sh.at[dst_v.at[b * KB + t]], add=True)

    fire_g(0, buf_a, sem_a)

    @pl.loop(0, NBLK - 2, step=2)
    def _(b):
      fire_g(b + 1, buf_b, sem_b)
      drain_g(buf_a, sem_a)
      scatter(b, buf_a)
      fire_g(b + 2, buf_a, sem_a)
      drain_g(buf_b, sem_b)
      scatter(b + 1, buf_b)

    fire_g(NBLK - 1, buf_b, sem_b)
    drain_g(buf_a, sem_a)
    scatter(NBLK - 2, buf_a)
    drain_g(buf_b, sem_b)
    scatter(NBLK - 1, buf_b)

    plsc.subcore_barrier()
    pltpu.sync_copy(acc_sh.at[pl.ds(s * RPS, RPS)],
                    out_hbm.at[c].at[pl.ds(s * RPS, RPS)])

  return k(table, src2d, dst2d, zeros)


def _tc_mm(x_pad, w1):
  def body(x_ref, w_ref, o_ref):
    o_ref[...] = jnp.dot(x_ref[...], w_ref[...],
                         preferred_element_type=jnp.float32)

  return pl.pallas_call(
      body,
      out_shape=jax.ShapeDtypeStruct((N_PAD, D_HID), jnp.float32),
  )(x_pad, w1)


def _tc_prep(hist, xw1):
  def body(h_ref, xw_ref, dis_ref, tab_ref):
    ones = jnp.ones((NW, D_HID), jnp.float32)
    deg = lax.dot_general(h_ref[...], ones, (((0,), (0,)), ((), ())),
                          preferred_element_type=jnp.float32) + 1.0
    dis = lax.rsqrt(deg)
    dis_ref[...] = dis
    tab_ref[...] = dis * xw_ref[...]

  return pl.pallas_call(
      body,
      out_shape=[
          jax.ShapeDtypeStruct((N_PAD, D_HID), jnp.float32),
          jax.ShapeDtypeStruct((N_PAD, D_HID), jnp.float32),
      ],
  )(hist, xw1)


def _tc_mid(p1, dis16, table1, b1):
  def body(p_ref, dis_ref, tab_ref, b_ref, o_ref):
    dis = dis_ref[...]
    h = jnp.maximum(dis * (p_ref[0] + p_ref[1] + tab_ref[...]) + b_ref[...],
                    0.0)
    o_ref[...] = dis * h

  return pl.pallas_call(
      body,
      out_shape=jax.ShapeDtypeStruct((N_PAD, D_HID), jnp.float32),
  )(p1, dis16, table1, b1)


def _tc_final(p2, dis16, table2, w2, b2):
  def body(p_ref, dis_ref, tab_ref, w_ref, b_ref, o_ref):
    ah = dis_ref[...] * (p_ref[0] + p_ref[1] + tab_ref[...])
    o_ref[...] = jnp.dot(ah, w_ref[...],
                         preferred_element_type=jnp.float32) + b_ref[...]

  return pl.pallas_call(
      body,
      out_shape=jax.ShapeDtypeStruct((N_PAD, D_OUT), jnp.float32),
  )(p2, dis16, table2, w2, b2)


@jax.jit
def kernel(x, edge_index, W1, b1, W2, b2):
  x_pad = jnp.pad(x, ((0, N_PAD - N_NODES), (0, 0)))
  pad = jnp.full((2, E_PAD - N_EDGES), N_PAD - 1, dtype=edge_index.dtype)
  edges = jnp.concatenate([edge_index, pad], axis=1)
  src2 = edges[0].reshape(NW * NCHUNK, CHUNK)
  dst2 = edges[1].reshape(NW * NCHUNK, CHUNK)
  z16 = jnp.zeros((N_PAD, D_HID), jnp.float32)

  xw1 = _tc_mm(x_pad, W1)
  hist = _sc_deg(edges[1]).reshape(NW, N_PAD)
  dis16, table1 = _tc_prep(hist, xw1)
  p1 = _sc_agg(table1, src2, dst2, z16, D_HID)
  table2 = _tc_mid(p1, dis16, table1, b1.reshape(1, D_HID))
  p2 = _sc_agg(table2, src2, dst2, z16, D_HID)
  out = _tc_final(p2, dis16, table2, W2, b2.reshape(1, -1))
  return out[:N_NODES]


# speedup vs baseline: 61.5945x; 1.3560x over previous
import functools

import jax
import jax.numpy as jnp
from jax import lax
from jax.experimental import pallas as pl
from jax.experimental.pallas import tpu as pltpu
from jax.experimental.pallas import tpu_sc as plsc

N_NODES = 10000
N_PAD = 10240
D_HID = 16
N_EDGES = 320000
NC = 2
NS = 16
NW = NC * NS
CHUNK = 128
NCHUNK = 80
EPT = CHUNK * NCHUNK
E_PAD = NW * EPT
RPS = N_PAD // NS
D_OUT = 3


def _sc_mesh():
  return plsc.VectorSubcoreMesh(core_axis_name="c", subcore_axis_name="s")


_SC_PARAMS = pltpu.CompilerParams(needs_layout_passes=False,
                                  use_tc_tiling_on_sc=False)


def _sc_deg(dst1d):
  @functools.partial(
      pl.kernel,
      out_type=jax.ShapeDtypeStruct((NW * N_PAD,), jnp.float32),
      mesh=_sc_mesh(),
      compiler_params=_SC_PARAMS,
      scratch_types=[
          pltpu.VMEM((EPT,), jnp.int32),
          pltpu.VMEM((N_PAD,), jnp.float32),
      ],
  )
  def k(dst_hbm, out_hbm, dst_v, hist_v):
    c = lax.axis_index("c")
    s = lax.axis_index("s")
    wid = c * NS + s
    zero16 = jnp.zeros((16,), jnp.float32)

    @pl.loop(0, N_PAD, step=16)
    def _(i):
      hist_v[pl.ds(i, 16)] = zero16

    pltpu.sync_copy(dst_hbm.at[pl.ds(wid * EPT, EPT)], dst_v)

    @pl.loop(0, EPT, step=16)
    def _(t):
      idx = dst_v[pl.ds(t, 16)]
      cnt, last = plsc.scan_count(idx)
      plsc.addupdate_scatter(hist_v, [idx], cnt.astype(jnp.float32), mask=last)

    pltpu.sync_copy(hist_v, out_hbm.at[pl.ds(wid * N_PAD, N_PAD)])

  return k(dst1d)


KB = 4
NBLK = NCHUNK // KB


def _sc_agg(table, src2d, dst2d, zeros, w):
  @functools.partial(
      pl.kernel,
      out_type=jax.ShapeDtypeStruct((NC, N_PAD, w), jnp.float32),
      mesh=_sc_mesh(),
      compiler_params=_SC_PARAMS,
      scratch_types=[
          pltpu.VMEM((NCHUNK, CHUNK), jnp.int32),
          pltpu.VMEM((NCHUNK, CHUNK), jnp.int32),
          pltpu.VMEM((KB, CHUNK, w), jnp.float32),
          pltpu.VMEM((KB, CHUNK, w), jnp.float32),
          pltpu.VMEM_SHARED((N_PAD, w), jnp.float32),
          pltpu.VMEM_SHARED((N_PAD, w), jnp.float32),
          pltpu.SemaphoreType.DMA,
          pltpu.SemaphoreType.DMA,
      ],
  )
  def k(tab_hbm, src_hbm, dst_hbm, z_hbm, out_hbm, src_v, dst_v, buf_a, buf_b,
        acc_sh, tab_sh, sem_a, sem_b):
    c = lax.axis_index("c")
    s = lax.axis_index("s")
    wid = c * NS + s

    pltpu.sync_copy(z_hbm.at[pl.ds(s * RPS, RPS)], acc_sh.at[pl.ds(s * RPS, RPS)])
    pltpu.sync_copy(tab_hbm.at[pl.ds(s * RPS, RPS)], tab_sh.at[pl.ds(s * RPS, RPS)])
    pltpu.sync_copy(src_hbm.at[pl.ds(wid * NCHUNK, NCHUNK)], src_v)
    pltpu.sync_copy(dst_hbm.at[pl.ds(wid * NCHUNK, NCHUNK)], dst_v)
    plsc.subcore_barrier()

    def fire_g(b, bufs, sem):
      for t in range(KB):
        pltpu.async_copy(tab_sh.at[src_v.at[b * KB + t]], bufs.at[t], sem)

    def drain_g(bufs, sem):
      for t in range(KB):
        pltpu.make_async_copy(tab_hbm.at[src_v.at[0]], bufs.at[t], sem).wait()

    def scatter(b, bufs):
      for t in range(KB):
        pltpu.sync_copy(bufs.at[t], acc_sh.at[dst_v.at[b * KB + t]], add=True)

    fire_g(0, buf_a, sem_a)

    @pl.loop(0, NBLK - 2, step=2)
    def _(b):
      fire_g(b + 1, buf_b, sem_b)
      drain_g(buf_a, sem_a)
      scatter(b, buf_a)
      fire_g(b + 2, buf_a, sem_a)
      drain_g(buf_b, sem_b)
      scatter(b + 1, buf_b)

    fire_g(NBLK - 1, buf_b, sem_b)
    drain_g(buf_a, sem_a)
    scatter(NBLK - 2, buf_a)
    drain_g(buf_b, sem_b)
    scatter(NBLK - 1, buf_b)

    plsc.subcore_barrier()
    pltpu.sync_copy(acc_sh.at[pl.ds(s * RPS, RPS)],
                    out_hbm.at[c].at[pl.ds(s * RPS, RPS)])

  return k(table, src2d, dst2d, zeros)


def _tc_mm(x_pad, w1):
  def body(x_ref, w_ref, o_ref):
    o_ref[...] = jnp.dot(x_ref[...], w_ref[...],
                         preferred_element_type=jnp.float32)

  return pl.pallas_call(
      body,
      out_shape=jax.ShapeDtypeStruct((N_PAD, D_HID), jnp.float32),
  )(x_pad, w1)


def _tc_prep(hist, xw1):
  def body(h_ref, xw_ref, dis_ref, tab_ref):
    ones = jnp.ones((NW, D_HID), jnp.float32)
    deg = lax.dot_general(h_ref[...], ones, (((0,), (0,)), ((), ())),
                          preferred_element_type=jnp.float32) + 1.0
    dis = lax.rsqrt(deg)
    dis_ref[...] = dis
    tab_ref[...] = dis * xw_ref[...]

  return pl.pallas_call(
      body,
      out_shape=[
          jax.ShapeDtypeStruct((N_PAD, D_HID), jnp.float32),
          jax.ShapeDtypeStruct((N_PAD, D_HID), jnp.float32),
      ],
  )(hist, xw1)


def _tc_mid(p1, dis16, table1, b1):
  def body(p_ref, dis_ref, tab_ref, b_ref, o_ref):
    dis = dis_ref[...]
    h = jnp.maximum(dis * (p_ref[0] + p_ref[1] + tab_ref[...]) + b_ref[...],
                    0.0)
    o_ref[...] = dis * h

  return pl.pallas_call(
      body,
      out_shape=jax.ShapeDtypeStruct((N_PAD, D_HID), jnp.float32),
  )(p1, dis16, table1, b1)


def _tc_final(p2, dis16, table2, w2, b2):
  def body(p_ref, dis_ref, tab_ref, w_ref, b_ref, o_ref):
    ah = dis_ref[...] * (p_ref[0] + p_ref[1] + tab_ref[...])
    o_ref[...] = jnp.dot(ah, w_ref[...],
                         preferred_element_type=jnp.float32) + b_ref[...]

  return pl.pallas_call(
      body,
      out_shape=jax.ShapeDtypeStruct((N_PAD, D_OUT), jnp.float32),
  )(p2, dis16, table2, w2, b2)


@jax.jit
def kernel(x, edge_index, W1, b1, W2, b2):
  x_pad = jnp.pad(x, ((0, N_PAD - N_NODES), (0, 0)))
  pad = jnp.full((2, E_PAD - N_EDGES), N_PAD - 1, dtype=edge_index.dtype)
  edges = jnp.concatenate([edge_index, pad], axis=1)
  src2 = edges[0].reshape(NW * NCHUNK, CHUNK)
  dst2 = edges[1].reshape(NW * NCHUNK, CHUNK)
  z16 = jnp.zeros((N_PAD, D_HID), jnp.float32)

  xw1 = _tc_mm(x_pad, W1)
  hist = _sc_deg(edges[1]).reshape(NW, N_PAD)
  dis16, table1 = _tc_prep(hist, xw1)
  p1 = _sc_agg(table1, src2, dst2, z16, D_HID)
  table2 = _tc_mid(p1, dis16, table1, b1.reshape(1, D_HID))
  p2 = _sc_agg(table2, src2, dst2, z16, D_HID)
  out = _tc_final(p2, dis16, table2, W2, b2.reshape(1, -1))
  return out[:N_NODES]


# speedup vs baseline: 66.3484x; 1.0772x over previous
import functools

import jax
import jax.numpy as jnp
from jax import lax
from jax.experimental import pallas as pl
from jax.experimental.pallas import tpu as pltpu
from jax.experimental.pallas import tpu_sc as plsc

N_NODES = 10000
N_PAD = 10240
D_IN = 128
D_HID = 16
D_OUT = 3
N_EDGES = 320000
NC = 2
NS = 16
NW = NC * NS
CHUNK = 128
NCHUNK = 80
EPT = CHUNK * NCHUNK
E_PAD = NW * EPT
EPT2 = E_PAD // NS
RPS = N_PAD // NS
VR = N_PAD // 8
KB = 4
NBLK = NCHUNK // KB


def _sc_mesh():
  return plsc.VectorSubcoreMesh(core_axis_name="c", subcore_axis_name="s")


_SC_PARAMS = pltpu.CompilerParams(needs_layout_passes=False,
                                  use_tc_tiling_on_sc=False)


def _rsqrt16(d):
  bits = plsc.bitcast(d, jnp.int32)
  y = plsc.bitcast(jnp.int32(0x5F3759DF) - (bits >> 1), jnp.float32)
  for _ in range(3):
    y = y * (1.5 - 0.5 * d * y * y)
  return y


def _sc_deg(dst1d):
  @functools.partial(
      pl.kernel,
      out_type=jax.ShapeDtypeStruct((NC, N_PAD), jnp.float32),
      mesh=_sc_mesh(),
      compiler_params=_SC_PARAMS,
      scratch_types=[
          pltpu.VMEM((EPT2,), jnp.int32),
          pltpu.VMEM((N_PAD,), jnp.float32),
          pltpu.VMEM((RPS,), jnp.float32),
          pltpu.VMEM((RPS,), jnp.float32),
          pltpu.VMEM_SHARED((NS, N_PAD), jnp.float32),
      ],
  )
  def k(dst_hbm, dis_hbm, dst_v, hist_v, tmp_v, acc_v, hist_sh):
    c = lax.axis_index("c")
    s = lax.axis_index("s")
    zero16 = jnp.zeros((16,), jnp.float32)

    @pl.loop(0, N_PAD, step=16)
    def _(i):
      hist_v[pl.ds(i, 16)] = zero16

    pltpu.sync_copy(dst_hbm.at[pl.ds(s * EPT2, EPT2)], dst_v)

    @pl.loop(0, EPT2, step=32)
    def _(t):
      for u in (0, 16):
        idx = dst_v[pl.ds(t + u, 16)]
        cnt, last = plsc.scan_count(idx)
        plsc.addupdate_scatter(hist_v, [idx], cnt.astype(jnp.float32),
                               mask=last)

    pltpu.sync_copy(hist_v, hist_sh.at[s])
    plsc.subcore_barrier()

    @pl.loop(0, RPS, step=16)
    def _(i):
      acc_v[pl.ds(i, 16)] = zero16

    for j in range(NS):
      pltpu.sync_copy(hist_sh.at[j].at[pl.ds(s * RPS, RPS)], tmp_v)

      @pl.loop(0, RPS, step=16)
      def _(i):
        acc_v[pl.ds(i, 16)] += tmp_v[pl.ds(i, 16)]

    @pl.loop(0, RPS, step=16)
    def _(i):
      acc_v[pl.ds(i, 16)] = _rsqrt16(acc_v[pl.ds(i, 16)] + 1.0)

    pltpu.sync_copy(acc_v, dis_hbm.at[c].at[pl.ds(s * RPS, RPS)])

  return k(dst1d)


def _sc_agg(table, src2d, dst2d, dis2, zeros):
  @functools.partial(
      pl.kernel,
      out_type=jax.ShapeDtypeStruct((NC, N_PAD, D_HID), jnp.float32),
      mesh=_sc_mesh(),
      compiler_params=_SC_PARAMS,
      scratch_types=[
          pltpu.VMEM((NCHUNK, CHUNK), jnp.int32),
          pltpu.VMEM((NCHUNK, CHUNK), jnp.int32),
          pltpu.VMEM((KB, CHUNK, D_HID), jnp.float32),
          pltpu.VMEM((KB, CHUNK, D_HID), jnp.float32),
          pltpu.VMEM((RPS, D_HID), jnp.float32),
          pltpu.VMEM((RPS, D_HID), jnp.float32),
          pltpu.VMEM((RPS,), jnp.float32),
          pltpu.VMEM_SHARED((N_PAD, D_HID), jnp.float32),
          pltpu.VMEM_SHARED((N_PAD, D_HID), jnp.float32),
          pltpu.SemaphoreType.DMA,
          pltpu.SemaphoreType.DMA,
      ],
  )
  def k(tab_hbm, src_hbm, dst_hbm, dis_hbm, z_hbm, out_hbm, src_v, dst_v,
        buf_a, buf_b, tab_v, acc_v, dis_v, acc_sh, tab_sh, sem_a, sem_b):
    c = lax.axis_index("c")
    s = lax.axis_index("s")
    wid = c * NS + s
    rows = pl.ds(s * RPS, RPS)

    pltpu.sync_copy(z_hbm.at[rows], acc_sh.at[rows])
    pltpu.sync_copy(tab_hbm.at[rows], tab_v)
    pltpu.sync_copy(dis_hbm.at[c].at[rows], dis_v)
    pltpu.sync_copy(src_hbm.at[pl.ds(wid * NCHUNK, NCHUNK)], src_v)
    pltpu.sync_copy(dst_hbm.at[pl.ds(wid * NCHUNK, NCHUNK)], dst_v)

    @pl.loop(0, RPS)
    def _(r):
      dr = plsc.load_gather(dis_v, [jnp.full((16,), r, jnp.int32)])
      tab_v[r, :] = tab_v[r, :] * dr

    pltpu.sync_copy(tab_v, tab_sh.at[rows])
    plsc.subcore_barrier()

    def fire_g(b, bufs, sem):
      for t in range(KB):
        pltpu.async_copy(tab_sh.at[src_v.at[b * KB + t]], bufs.at[t], sem)

    def drain_g(bufs, sem):
      for t in range(KB):
        pltpu.make_async_copy(tab_hbm.at[src_v.at[0]], bufs.at[t], sem).wait()

    def scatter(b, bufs):
      for t in range(KB):
        pltpu.sync_copy(bufs.at[t], acc_sh.at[dst_v.at[b * KB + t]], add=True)

    fire_g(0, buf_a, sem_a)

    @pl.loop(0, NBLK - 2, step=2)
    def _(b):
      fire_g(b + 1, buf_b, sem_b)
      drain_g(buf_a, sem_a)
      scatter(b, buf_a)
      fire_g(b + 2, buf_a, sem_a)
      drain_g(buf_b, sem_b)
      scatter(b + 1, buf_b)

    fire_g(NBLK - 1, buf_b, sem_b)
    drain_g(buf_a, sem_a)
    scatter(NBLK - 2, buf_a)
    drain_g(buf_b, sem_b)
    scatter(NBLK - 1, buf_b)

    plsc.subcore_barrier()

    pltpu.sync_copy(acc_sh.at[rows], acc_v)
    f16 = jnp.where(jnp.broadcast_to(c, (16,)) == 0, 1.0, 0.0)

    @pl.loop(0, RPS)
    def _(r):
      dr = plsc.load_gather(dis_v, [jnp.full((16,), r, jnp.int32)])
      acc_v[r, :] = (acc_v[r, :] + f16 * tab_v[r, :]) * dr

    pltpu.sync_copy(acc_v, out_hbm.at[c].at[rows])

  return k(table, src2d, dst2d, dis2, zeros)


def _tc_mm(xv, w1b):
  def body(x_ref, w_ref, o_ref):
    o_ref[...] = jnp.dot(x_ref[...], w_ref[...],
                         preferred_element_type=jnp.float32)

  return pl.pallas_call(
      body,
      out_shape=jax.ShapeDtypeStruct((VR, 128), jnp.float32),
  )(xv, w1b)


def _tc_mid(q1, b1t):
  def body(q_ref, b_ref, o_ref):
    o_ref[...] = jnp.maximum(q_ref[0] + q_ref[1] + b_ref[...], 0.0)

  return pl.pallas_call(
      body,
      out_shape=jax.ShapeDtypeStruct((VR, 128), jnp.float32),
  )(q1, b1t)


def _tc_final(q2, w2b, b2t):
  def body(q_ref, w_ref, b_ref, o_ref):
    ah = q_ref[0] + q_ref[1]
    o_ref[...] = jnp.dot(ah, w_ref[...],
                         preferred_element_type=jnp.float32) + b_ref[...]

  return pl.pallas_call(
      body,
      out_shape=jax.ShapeDtypeStruct((VR, 8 * D_OUT), jnp.float32),
  )(q2, w2b, b2t)


def _block_diag(w, n):
  d0, d1 = w.shape
  out = jnp.zeros((n * d0, n * d1), w.dtype)
  for i in range(n):
    out = out.at[i * d0:(i + 1) * d0, i * d1:(i + 1) * d1].set(w)
  return out


@jax.jit
def kernel(x, edge_index, W1, b1, W2, b2):
  x_pad = jnp.pad(x, ((0, N_PAD - N_NODES), (0, 0)))
  xv = x_pad.reshape(VR, 8 * D_IN)
  pad = jnp.full((2, E_PAD - N_EDGES), N_PAD - 1, dtype=edge_index.dtype)
  edges = jnp.concatenate([edge_index, pad], axis=1)
  src2 = edges[0].reshape(NW * NCHUNK, CHUNK)
  dst2 = edges[1].reshape(NW * NCHUNK, CHUNK)
  z16 = jnp.zeros((N_PAD, D_HID), jnp.float32)
  w1b = _block_diag(W1, 8)
  w2b = _block_diag(W2, 8)
  b1t = jnp.tile(b1, 8).reshape(1, 128)
  b2t = jnp.tile(b2, 8).reshape(1, 8 * D_OUT)

  xw1 = _tc_mm(xv, w1b)
  dis2 = _sc_deg(edges[1])
  q1 = _sc_agg(xw1.reshape(N_PAD, D_HID), src2, dst2, dis2, z16)
  h = _tc_mid(q1.reshape(NC, VR, 128), b1t)
  q2 = _sc_agg(h.reshape(N_PAD, D_HID), src2, dst2, dis2, z16)
  out = _tc_final(q2.reshape(NC, VR, 128), w2b, b2t)
  return out.reshape(N_PAD, D_OUT)[:N_NODES]
